# Initial kernel scaffold; baseline (speedup 1.0000x reference)
import jax
import jax.numpy as jnp
from jax.experimental import pallas as pl


def kernel(x, freqs_cos, freqs_sin, wq_a, q_norm_w, wq_b, wkv_a, kv_norm_w, wkv_b, wo, idx_wq, idx_wk, idx_ln_w, idx_ln_b, idx_ww):
    raise NotImplementedError("write your pallas kernel here")


# speedup vs baseline: 10.7915x; 10.7915x over previous
import functools

import jax
import jax.numpy as jnp
import numpy as np
from jax.experimental import pallas as pl
from jax.experimental.pallas import tpu as pltpu

_H = 16
_DN, _DR, _DV = 128, 64, 128
_DQK = _DN + _DR
_IDX_H, _IDX_D = 32, 128
_TOPK = 1024
_NEG = -1e9


def _deint(n):
    return np.concatenate([np.arange(0, n, 2), np.arange(1, n, 2)])


def _rope_deint(x, c, s):
    xr = x[:, : _DR // 2]
    xi = x[:, _DR // 2 :]
    return jnp.concatenate([xr * c - xi * s, xr * s + xi * c], axis=1)


def _stage1_body(x_ref, wqaT_ref, qnw_ref, wkvaT_ref, kvnw_ref, wkiT_ref,
                 lnw_ref, lnb_ref, cos_ref, sin_ref,
                 cq_ref, ckv_ref, krope_ref, ki_ref):
    xb = x_ref[...]
    c = cos_ref[...]
    s = sin_ref[...]

    t = jnp.dot(xb, wqaT_ref[...], preferred_element_type=jnp.float32)
    cq_ref[...] = t * jax.lax.rsqrt(
        jnp.mean(t * t, axis=1, keepdims=True) + 1e-6) * qnw_ref[...]

    kv = jnp.dot(xb, wkvaT_ref[...], preferred_element_type=jnp.float32)
    ck = kv[:, :512]
    ckv_ref[...] = ck * jax.lax.rsqrt(
        jnp.mean(ck * ck, axis=1, keepdims=True) + 1e-6) * kvnw_ref[...]
    krope_ref[...] = _rope_deint(kv[:, 512:], c, s)

    kt = jnp.dot(xb, wkiT_ref[...], preferred_element_type=jnp.float32)
    m = jnp.mean(kt, axis=1, keepdims=True)
    v = jnp.mean((kt - m) * (kt - m), axis=1, keepdims=True)
    kn = (kt - m) * jax.lax.rsqrt(v + 1e-5) * lnw_ref[...] + lnb_ref[...]
    ki_ref[...] = jnp.concatenate(
        [_rope_deint(kn[:, :_DR], c, s), kn[:, _DR:]], axis=1)


def _stage3_body(cq_ref, x_ref, wqiT_ref, wwT_ref, ki_ref, cos_ref, sin_ref,
                 mask_ref, *, bm, seq, k):
    i = pl.program_id(0)
    c = cos_ref[...]
    s = sin_ref[...]
    wts = jnp.dot(x_ref[...], wwT_ref[...],
                  preferred_element_type=jnp.float32) * (_IDX_H ** -0.5)
    qi = jnp.dot(cq_ref[...], wqiT_ref[...],
                 preferred_element_type=jnp.float32)
    kif = ki_ref[...]
    sc = jnp.zeros((bm, seq), jnp.float32)
    for h in range(_IDX_H):
        qih = qi[:, h * _IDX_D:(h + 1) * _IDX_D]
        qih = jnp.concatenate([_rope_deint(qih[:, :_DR], c, s), qih[:, _DR:]],
                              axis=1)
        sh = jax.lax.dot_general(qih, kif, (((1,), (1,)), ((), ())),
                                 preferred_element_type=jnp.float32)
        sc = sc + (sh * (_IDX_D ** -0.5)) * wts[:, h:h + 1]
    row = i * bm + jax.lax.broadcasted_iota(jnp.int32, (bm, seq), 0)
    col = jax.lax.broadcasted_iota(jnp.int32, (bm, seq), 1)
    sc = sc + jnp.where(col <= row, 0.0, _NEG)

    u = jax.lax.bitcast_convert_type(sc, jnp.uint32)
    sign = u >= jnp.uint32(0x80000000)
    su = jnp.where(sign, ~u, u | jnp.uint32(0x80000000))

    T = jnp.zeros((bm, 1), jnp.uint32)
    for b in range(31, -1, -1):
        Tt = T | jnp.uint32(2 ** b)
        cnt = jnp.sum((su >= Tt).astype(jnp.int32), axis=1, keepdims=True)
        T = jnp.where(cnt >= k, Tt, T)
    mask_ref[...] = (su >= T).astype(jnp.int8)


def _stage4a_body(cq_ref, wn_ref, wr_ref, cos_ref, sin_ref, qn_ref, qr_ref):
    cqb = cq_ref[...]
    c = cos_ref[...]
    s = sin_ref[...]
    for h in range(_H):
        qn_ref[h] = jnp.dot(cqb, wn_ref[h], preferred_element_type=jnp.float32)
        qr_ref[h] = _rope_deint(
            jnp.dot(cqb, wr_ref[h], preferred_element_type=jnp.float32), c, s)


def _stage4b_body(ckv_ref, wk_ref, wv_ref, kn_ref, v_ref):
    ckvb = ckv_ref[...]
    for h in range(_H):
        kn_ref[h] = jnp.dot(ckvb, wk_ref[h], preferred_element_type=jnp.float32)
        v_ref[h] = jnp.dot(ckvb, wv_ref[h], preferred_element_type=jnp.float32)


def _stage5_body(qn_ref, qr_ref, kn_ref, v_ref, krope_ref, mask_ref,
                 out_ref, *, bm, seq):
    i = pl.program_id(1)
    att = jax.lax.dot_general(qn_ref[0], kn_ref[0], (((1,), (1,)), ((), ())),
                              preferred_element_type=jnp.float32)
    att = att + jax.lax.dot_general(qr_ref[0], krope_ref[...],
                                    (((1,), (1,)), ((), ())),
                                    preferred_element_type=jnp.float32)
    att = att * (_DQK ** -0.5)

    row = i * bm + jax.lax.broadcasted_iota(jnp.int32, (bm, seq), 0)
    col = jax.lax.broadcasted_iota(jnp.int32, (bm, seq), 1)
    keep = jnp.logical_and(col <= row, mask_ref[...] != 0)

    att = jnp.where(keep, att, _NEG)
    m = jnp.max(att, axis=1, keepdims=True)
    p = jnp.exp(att - m)
    p = p / jnp.sum(p, axis=1, keepdims=True)
    out_ref[0] = jnp.dot(p, v_ref[0], preferred_element_type=jnp.float32)


def _stage6_body(attn_ref, woT_ref, out_ref):
    acc = jnp.dot(attn_ref[0], woT_ref[0], preferred_element_type=jnp.float32)
    for h in range(1, _H):
        acc = acc + jnp.dot(attn_ref[h], woT_ref[h],
                            preferred_element_type=jnp.float32)
    out_ref[...] = acc


def kernel(x, freqs_cos, freqs_sin, wq_a, q_norm_w, wq_b, wkv_a, kv_norm_w,
           wkv_b, wo, idx_wq, idx_wk, idx_ln_w, idx_ln_b, idx_ww):
    b, seq, dm = x.shape
    x2 = x[0]
    bm = 256
    nblk = seq // bm
    k = min(_TOPK, seq)
    q_lora = wq_a.shape[0]
    kv_lora = kv_norm_w.shape[0]

    pd = _deint(_DR)

    wqaT = wq_a.T
    qnw = q_norm_w[None, :]
    wkvaT = wkv_a.T
    wkvaT = jnp.concatenate([wkvaT[:, :kv_lora], wkvaT[:, kv_lora:][:, pd]], 1)
    kvnw = kv_norm_w[None, :]
    pki = np.concatenate([pd, np.arange(_DR, _IDX_D)])
    wkiT = idx_wk.T[:, pki]
    lnw = idx_ln_w[None, pki]
    lnb = idx_ln_b[None, pki]
    wqi3 = idx_wq.reshape(_IDX_H, _IDX_D, q_lora)[:, pki, :]
    wqiT = wqi3.reshape(_IDX_H * _IDX_D, q_lora).T
    wwT = idx_ww.T
    wqb3 = wq_b.reshape(_H, _DQK, q_lora)
    wqbT_n = jnp.transpose(wqb3[:, :_DN, :], (0, 2, 1))
    wqbT_r = jnp.transpose(wqb3[:, _DN:, :][:, pd, :], (0, 2, 1))
    wkvb3 = wkv_b.reshape(_H, _DN + _DV, kv_lora)
    wkvbT_k = jnp.transpose(wkvb3[:, :_DN, :], (0, 2, 1))
    wkvbT_v = jnp.transpose(wkvb3[:, _DN:, :], (0, 2, 1))
    woT3 = wo.T.reshape(_H, _DV, dm)

    f32 = jnp.float32
    row_spec = lambda w: pl.BlockSpec((bm, w), lambda i: (i, 0))
    full_spec = lambda a: pl.BlockSpec(a.shape, lambda *_: (0,) * a.ndim)

    cq, ckv, krope, ki = pl.pallas_call(
        _stage1_body,
        grid=(nblk,),
        in_specs=[row_spec(dm), full_spec(wqaT), full_spec(qnw),
                  full_spec(wkvaT), full_spec(kvnw), full_spec(wkiT),
                  full_spec(lnw), full_spec(lnb),
                  row_spec(_DR // 2), row_spec(_DR // 2)],
        out_specs=[row_spec(q_lora), row_spec(kv_lora), row_spec(_DR),
                   row_spec(_IDX_D)],
        out_shape=[jax.ShapeDtypeStruct((seq, q_lora), f32),
                   jax.ShapeDtypeStruct((seq, kv_lora), f32),
                   jax.ShapeDtypeStruct((seq, _DR), f32),
                   jax.ShapeDtypeStruct((seq, _IDX_D), f32)],
    )(x2, wqaT, qnw, wkvaT, kvnw, wkiT, lnw, lnb, freqs_cos, freqs_sin)

    mask = pl.pallas_call(
        functools.partial(_stage3_body, bm=bm, seq=seq, k=k),
        grid=(nblk,),
        in_specs=[row_spec(q_lora), row_spec(dm), full_spec(wqiT),
                  full_spec(wwT), full_spec(ki),
                  row_spec(_DR // 2), row_spec(_DR // 2)],
        out_specs=row_spec(seq),
        out_shape=jax.ShapeDtypeStruct((seq, seq), jnp.int8),
    )(cq, x2, wqiT, wwT, ki, freqs_cos, freqs_sin)

    head_row = lambda w: pl.BlockSpec((_H, bm, w), lambda i: (0, i, 0))
    qn, qr = pl.pallas_call(
        _stage4a_body,
        grid=(nblk,),
        in_specs=[row_spec(q_lora), full_spec(wqbT_n), full_spec(wqbT_r),
                  row_spec(_DR // 2), row_spec(_DR // 2)],
        out_specs=[head_row(_DN), head_row(_DR)],
        out_shape=[jax.ShapeDtypeStruct((_H, seq, _DN), f32),
                   jax.ShapeDtypeStruct((_H, seq, _DR), f32)],
    )(cq, wqbT_n, wqbT_r, freqs_cos, freqs_sin)

    kn, v = pl.pallas_call(
        _stage4b_body,
        grid=(nblk,),
        in_specs=[row_spec(kv_lora), full_spec(wkvbT_k), full_spec(wkvbT_v)],
        out_specs=[head_row(_DN), head_row(_DV)],
        out_shape=[jax.ShapeDtypeStruct((_H, seq, _DN), f32),
                   jax.ShapeDtypeStruct((_H, seq, _DV), f32)],
    )(ckv, wkvbT_k, wkvbT_v)

    hblk = lambda w: pl.BlockSpec((1, bm, w), lambda h, i: (h, i, 0))
    hfull = lambda w: pl.BlockSpec((1, seq, w), lambda h, i: (h, 0, 0))
    attn = pl.pallas_call(
        functools.partial(_stage5_body, bm=bm, seq=seq),
        grid=(_H, nblk),
        in_specs=[hblk(_DN), hblk(_DR), hfull(_DN), hfull(_DV),
                  pl.BlockSpec((seq, _DR), lambda h, i: (0, 0)),
                  pl.BlockSpec((bm, seq), lambda h, i: (i, 0))],
        out_specs=hblk(_DV),
        out_shape=jax.ShapeDtypeStruct((_H, seq, _DV), f32),
    )(qn, qr, kn, v, krope, mask)

    out = pl.pallas_call(
        _stage6_body,
        grid=(nblk,),
        in_specs=[head_row(_DV), full_spec(woT3)],
        out_specs=row_spec(dm),
        out_shape=jax.ShapeDtypeStruct((seq, dm), f32),
    )(attn, woT3)

    return out[None]


# speedup vs baseline: 15.7643x; 1.4608x over previous
import functools

import jax
import jax.numpy as jnp
import numpy as np
from jax.experimental import pallas as pl
from jax.experimental.pallas import tpu as pltpu

_H = 16
_DN, _DR, _DV = 128, 64, 128
_DQK = _DN + _DR
_IDX_H, _IDX_D = 32, 128
_TOPK = 1024
_NEG = -1e9


def _dotT(a, b):
    return jax.lax.dot_general(a, b, (((1,), (1,)), ((), ())),
                               preferred_element_type=jnp.float32)


def _rope_int(x, c2, s2):
    n = x.shape[1]
    even = jax.lax.broadcasted_iota(jnp.int32, x.shape, 1) % 2 == 0
    up = pltpu.roll(x, n - 1, 1)
    dn = pltpu.roll(x, 1, 1)
    rot = jnp.where(even, -up, dn)
    return x * c2 + rot * s2


def _stage1_body(x_ref, wqa_ref, qnw_ref, wkva_ref, kvnw_ref, wki_ref,
                 lnw_ref, lnb_ref, cos_ref, sin_ref,
                 cq_ref, ckv_ref, krope_ref, ki_ref):
    xb = x_ref[...]
    c2 = cos_ref[...]
    s2 = sin_ref[...]

    t = _dotT(xb, wqa_ref[...])
    cq_ref[...] = t * jax.lax.rsqrt(
        jnp.mean(t * t, axis=1, keepdims=True) + 1e-6) * qnw_ref[...]

    kv = _dotT(xb, wkva_ref[...])
    ck = kv[:, :512]
    ckv_ref[...] = ck * jax.lax.rsqrt(
        jnp.mean(ck * ck, axis=1, keepdims=True) + 1e-6) * kvnw_ref[...]
    krope_ref[...] = _rope_int(kv[:, 512:], c2, s2)

    kt = _dotT(xb, wki_ref[...])
    m = jnp.mean(kt, axis=1, keepdims=True)
    v = jnp.mean((kt - m) * (kt - m), axis=1, keepdims=True)
    kn = (kt - m) * jax.lax.rsqrt(v + 1e-5) * lnw_ref[...] + lnb_ref[...]
    ki_ref[...] = jnp.concatenate(
        [_rope_int(kn[:, :_DR], c2, s2), kn[:, _DR:]], axis=1)


def _stage3_body(cq_ref, x_ref, wqi_ref, ww_ref, ki_ref, cos_ref, sin_ref,
                 mask_ref, *, bm, seq, k):
    i = pl.program_id(0)

    @pl.when(i * bm + bm <= k)
    def _all_keep():
        mask_ref[...] = jnp.ones((bm, seq), jnp.int8)

    @pl.when(i * bm + bm > k)
    def _select():
        c2 = cos_ref[...]
        s2 = sin_ref[...]
        wts = _dotT(x_ref[...], ww_ref[...]) * (_IDX_H ** -0.5)
        qi = _dotT(cq_ref[...], wqi_ref[...])
        kif = ki_ref[...]
        sc = jnp.zeros((bm, seq), jnp.float32)
        for h in range(_IDX_H):
            qih = qi[:, h * _IDX_D:(h + 1) * _IDX_D]
            qih = jnp.concatenate(
                [_rope_int(qih[:, :_DR], c2, s2), qih[:, _DR:]], axis=1)
            sc = sc + (_dotT(qih, kif) * (_IDX_D ** -0.5)) * wts[:, h:h + 1]
        row = i * bm + jax.lax.broadcasted_iota(jnp.int32, (bm, seq), 0)
        col = jax.lax.broadcasted_iota(jnp.int32, (bm, seq), 1)
        sc = sc + jnp.where(col <= row, 0.0, _NEG)

        u = jax.lax.bitcast_convert_type(sc, jnp.uint32)
        sign = u >= jnp.uint32(0x80000000)
        su = jnp.where(sign, ~u, u | jnp.uint32(0x80000000))

        T = jnp.zeros((bm, 1), jnp.uint32)
        for b in range(31, -1, -1):
            Tt = T | jnp.uint32(2 ** b)
            cnt = jnp.sum((su >= Tt).astype(jnp.int32), axis=1, keepdims=True)
            T = jnp.where(cnt >= k, Tt, T)
        mask_ref[...] = (su >= T).astype(jnp.int8)


def _stage4a_body(cq_ref, wqb_ref, cos_ref, sin_ref, qn_ref, qr_ref):
    cqb = cq_ref[...]
    c2 = cos_ref[...]
    s2 = sin_ref[...]
    for h in range(_H):
        q = _dotT(cqb, wqb_ref[h * _DQK:(h + 1) * _DQK, :])
        qn_ref[h] = q[:, :_DN]
        qr_ref[h] = _rope_int(q[:, _DN:], c2, s2)


def _stage4b_body(ckv_ref, wkvb_ref, kn_ref, v_ref):
    ckvb = ckv_ref[...]
    for h in range(_H):
        kvh = _dotT(ckvb, wkvb_ref[h * (_DN + _DV):(h + 1) * (_DN + _DV), :])
        kn_ref[h] = kvh[:, :_DN]
        v_ref[h] = kvh[:, _DN:]


def _stage5_body(qn_ref, qr_ref, kn_ref, v_ref, krope_ref, mask_ref,
                 out_ref, *, bm, seq):
    i = pl.program_id(1)
    att = _dotT(qn_ref[0], kn_ref[0]) + _dotT(qr_ref[0], krope_ref[...])
    att = att * (_DQK ** -0.5)

    row = i * bm + jax.lax.broadcasted_iota(jnp.int32, (bm, seq), 0)
    col = jax.lax.broadcasted_iota(jnp.int32, (bm, seq), 1)
    keep = jnp.logical_and(col <= row, mask_ref[...] != 0)

    att = jnp.where(keep, att, _NEG)
    m = jnp.max(att, axis=1, keepdims=True)
    p = jnp.exp(att - m)
    p = p / jnp.sum(p, axis=1, keepdims=True)
    out_ref[0] = jnp.dot(p, v_ref[0], preferred_element_type=jnp.float32)


def _stage6_body(attn_ref, wo_ref, out_ref):
    acc = _dotT(attn_ref[0], wo_ref[:, :_DV])
    for h in range(1, _H):
        acc = acc + _dotT(attn_ref[h], wo_ref[:, h * _DV:(h + 1) * _DV])
    out_ref[...] = acc


def kernel(x, freqs_cos, freqs_sin, wq_a, q_norm_w, wq_b, wkv_a, kv_norm_w,
           wkv_b, wo, idx_wq, idx_wk, idx_ln_w, idx_ln_b, idx_ww):
    b, seq, dm = x.shape
    x2 = x[0]
    bm = 256
    nblk = seq // bm
    k = min(_TOPK, seq)
    q_lora = wq_a.shape[0]
    kv_lora = kv_norm_w.shape[0]

    c2 = jnp.repeat(freqs_cos, 2, axis=1)
    s2 = jnp.repeat(freqs_sin, 2, axis=1)
    qnw = q_norm_w[None, :]
    kvnw = kv_norm_w[None, :]
    lnw = idx_ln_w[None, :]
    lnb = idx_ln_b[None, :]

    f32 = jnp.float32
    row_spec = lambda w: pl.BlockSpec((bm, w), lambda i: (i, 0))
    full_spec = lambda a: pl.BlockSpec(a.shape, lambda *_: (0,) * a.ndim)

    cq, ckv, krope, ki = pl.pallas_call(
        _stage1_body,
        grid=(nblk,),
        in_specs=[row_spec(dm), full_spec(wq_a), full_spec(qnw),
                  full_spec(wkv_a), full_spec(kvnw), full_spec(idx_wk),
                  full_spec(lnw), full_spec(lnb),
                  row_spec(_DR), row_spec(_DR)],
        out_specs=[row_spec(q_lora), row_spec(kv_lora), row_spec(_DR),
                   row_spec(_IDX_D)],
        out_shape=[jax.ShapeDtypeStruct((seq, q_lora), f32),
                   jax.ShapeDtypeStruct((seq, kv_lora), f32),
                   jax.ShapeDtypeStruct((seq, _DR), f32),
                   jax.ShapeDtypeStruct((seq, _IDX_D), f32)],
    )(x2, wq_a, qnw, wkv_a, kvnw, idx_wk, lnw, lnb, c2, s2)

    mask = pl.pallas_call(
        functools.partial(_stage3_body, bm=bm, seq=seq, k=k),
        grid=(nblk,),
        in_specs=[row_spec(q_lora), row_spec(dm), full_spec(idx_wq),
                  full_spec(idx_ww), full_spec(ki),
                  row_spec(_DR), row_spec(_DR)],
        out_specs=row_spec(seq),
        out_shape=jax.ShapeDtypeStruct((seq, seq), jnp.int8),
    )(cq, x2, idx_wq, idx_ww, ki, c2, s2)

    head_row = lambda w: pl.BlockSpec((_H, bm, w), lambda i: (0, i, 0))
    qn, qr = pl.pallas_call(
        _stage4a_body,
        grid=(nblk,),
        in_specs=[row_spec(q_lora), full_spec(wq_b),
                  row_spec(_DR), row_spec(_DR)],
        out_specs=[head_row(_DN), head_row(_DR)],
        out_shape=[jax.ShapeDtypeStruct((_H, seq, _DN), f32),
                   jax.ShapeDtypeStruct((_H, seq, _DR), f32)],
    )(cq, wq_b, c2, s2)

    kn, v = pl.pallas_call(
        _stage4b_body,
        grid=(nblk,),
        in_specs=[row_spec(kv_lora), full_spec(wkv_b)],
        out_specs=[head_row(_DN), head_row(_DV)],
        out_shape=[jax.ShapeDtypeStruct((_H, seq, _DN), f32),
                   jax.ShapeDtypeStruct((_H, seq, _DV), f32)],
    )(ckv, wkv_b)

    hblk = lambda w: pl.BlockSpec((1, bm, w), lambda h, i: (h, i, 0))
    hfull = lambda w: pl.BlockSpec((1, seq, w), lambda h, i: (h, 0, 0))
    attn = pl.pallas_call(
        functools.partial(_stage5_body, bm=bm, seq=seq),
        grid=(_H, nblk),
        in_specs=[hblk(_DN), hblk(_DR), hfull(_DN), hfull(_DV),
                  pl.BlockSpec((seq, _DR), lambda h, i: (0, 0)),
                  pl.BlockSpec((bm, seq), lambda h, i: (i, 0))],
        out_specs=hblk(_DV),
        out_shape=jax.ShapeDtypeStruct((_H, seq, _DV), f32),
    )(qn, qr, kn, v, krope, mask)

    out = pl.pallas_call(
        _stage6_body,
        grid=(nblk,),
        in_specs=[head_row(_DV), full_spec(wo)],
        out_specs=row_spec(dm),
        out_shape=jax.ShapeDtypeStruct((seq, dm), f32),
    )(attn, wo)

    return out[None]
